# baseline (speedup 1.0000x reference)
import jax
import jax.numpy as jnp
from jax.experimental import pallas as pl
from jax.experimental.pallas import tpu as pltpu

B, L, D = 4096, 200, 128
N = B * L
G = N // 128
RBm = 128


def _body(mask_ref, w_ref, out_ref):
    m = mask_ref[...]
    w0 = w_ref[0]
    w1 = w_ref[1]
    m3 = jax.lax.broadcast_in_dim(m, (RBm, 128, D), (0, 1))
    out_ref[...] = jnp.where(m3 != 0, w1[None, None, :], w0[None, None, :])


def kernel(is_target_mask, embedding_weight):
    mask_packed = is_target_mask.astype(jnp.int32).reshape(G, 128)
    grid = (G // RBm,)
    out = pl.pallas_call(
        _body,
        grid=grid,
        in_specs=[
            pl.BlockSpec((RBm, 128), lambda i: (i, 0)),
            pl.BlockSpec((2, D), lambda i: (0, 0)),
        ],
        out_specs=pl.BlockSpec((RBm, 128, D), lambda i: (i, 0, 0)),
        out_shape=jax.ShapeDtypeStruct((G, 128, D), jnp.float32),
        compiler_params=pltpu.CompilerParams(
            dimension_semantics=("parallel",),
        ),
    )(mask_packed, embedding_weight)
    return out.reshape(B, L, D)


# speedup vs baseline: 1.0347x; 1.0347x over previous
import jax
import jax.numpy as jnp
from jax.experimental import pallas as pl
from jax.experimental.pallas import tpu as pltpu

B, L, D = 4096, 200, 128
N = B * L
G = N // 128
RBm = 320


def _body(mask_ref, w_ref, out_ref):
    m = mask_ref[...]
    w0 = w_ref[0]
    w1 = w_ref[1]
    m3 = jax.lax.broadcast_in_dim(m, (RBm, 128, D), (0, 1))
    out_ref[...] = jnp.where(m3 != 0, w1[None, None, :], w0[None, None, :])


def kernel(is_target_mask, embedding_weight):
    mask_packed = is_target_mask.astype(jnp.int32).reshape(G, 128)
    grid = (G // RBm,)
    out = pl.pallas_call(
        _body,
        grid=grid,
        in_specs=[
            pl.BlockSpec((RBm, 128), lambda i: (i, 0)),
            pl.BlockSpec((2, D), lambda i: (0, 0)),
        ],
        out_specs=pl.BlockSpec((RBm, 128, D), lambda i: (i, 0, 0)),
        out_shape=jax.ShapeDtypeStruct((G, 128, D), jnp.float32),
        compiler_params=pltpu.CompilerParams(
            dimension_semantics=("parallel",),
        ),
    )(mask_packed, embedding_weight)
    return out.reshape(B, L, D)
